# baseline (speedup 1.0000x reference)
import functools

import jax
import jax.numpy as jnp
from jax import lax
from jax.experimental import pallas as pl
from jax.experimental.pallas import tpu as pltpu
from jax.experimental.pallas import tpu_sc as plsc

N_ROWS = 16384
N_FEAT = 2048
BLOCK_ROWS = 16


def kernel(x, indices):
    info = plsc.get_sparse_core_info()
    num_cores, num_subcores, lanes = (
        info.num_cores, info.num_subcores, info.num_lanes)
    num_workers = num_cores * num_subcores
    rows_per_worker = N_ROWS // num_workers
    num_blocks = rows_per_worker // BLOCK_ROWS
    block_words = BLOCK_ROWS * N_FEAT
    mesh = plsc.VectorSubcoreMesh(core_axis_name="c", subcore_axis_name="s")

    @functools.partial(
        pl.kernel,
        mesh=mesh,
        compiler_params=pltpu.CompilerParams(needs_layout_passes=False),
        out_type=jax.ShapeDtypeStruct((N_ROWS * N_FEAT,), jnp.float32),
        scratch_types=[
            pltpu.VMEM((N_FEAT,), jnp.int32),
            pltpu.VMEM((block_words,), jnp.float32),
            pltpu.VMEM((block_words,), jnp.float32),
        ],
    )
    def permute_rows(x_hbm, idx_hbm, out_hbm, idx_v, in_v, out_v):
        wid = lax.axis_index("s") * num_cores + lax.axis_index("c")
        base = wid * rows_per_worker
        pltpu.sync_copy(idx_hbm, idx_v)

        def block_body(b, carry):
            elem0 = (base + b * BLOCK_ROWS) * N_FEAT
            pltpu.sync_copy(x_hbm.at[pl.ds(elem0, block_words)], in_v)

            def col_body(jb, c):
                col0 = jb * lanes
                idxv = idx_v[pl.ds(col0, lanes)]
                for r in range(BLOCK_ROWS):
                    vals = plsc.load_gather(in_v, [idxv + r * N_FEAT])
                    out_v[pl.ds(r * N_FEAT + col0, lanes)] = vals
                return c

            lax.fori_loop(0, N_FEAT // lanes, col_body, 0)
            pltpu.sync_copy(out_v, out_hbm.at[pl.ds(elem0, block_words)])
            return carry

        lax.fori_loop(0, num_blocks, block_body, 0)

    out_flat = permute_rows(x.reshape(-1), indices)
    return out_flat.reshape(N_ROWS, N_FEAT)


# speedup vs baseline: 1.1557x; 1.1557x over previous
import functools

import jax
import jax.numpy as jnp
from jax import lax
from jax.experimental import pallas as pl
from jax.experimental.pallas import tpu as pltpu
from jax.experimental.pallas import tpu_sc as plsc

N_ROWS = 16384
N_FEAT = 2048
BLOCK_ROWS = 8


def kernel(x, indices):
    info = plsc.get_sparse_core_info()
    num_cores, num_subcores, lanes = (
        info.num_cores, info.num_subcores, info.num_lanes)
    num_workers = num_cores * num_subcores
    rows_per_worker = N_ROWS // num_workers
    num_blocks = rows_per_worker // BLOCK_ROWS
    num_pairs = num_blocks // 2
    block_words = BLOCK_ROWS * N_FEAT
    mesh = plsc.VectorSubcoreMesh(core_axis_name="c", subcore_axis_name="s")

    @functools.partial(
        pl.kernel,
        mesh=mesh,
        compiler_params=pltpu.CompilerParams(needs_layout_passes=False),
        out_type=jax.ShapeDtypeStruct((N_ROWS * N_FEAT,), jnp.float32),
        scratch_types=[
            pltpu.VMEM((N_FEAT,), jnp.int32),
            pltpu.VMEM((block_words,), jnp.float32),
            pltpu.VMEM((block_words,), jnp.float32),
            pltpu.VMEM((block_words,), jnp.float32),
            pltpu.VMEM((block_words,), jnp.float32),
            pltpu.SemaphoreType.DMA,
            pltpu.SemaphoreType.DMA,
            pltpu.SemaphoreType.DMA,
            pltpu.SemaphoreType.DMA,
        ],
    )
    def permute_rows(x_hbm, idx_hbm, out_hbm, idx_v,
                     in_v0, in_v1, out_v0, out_v1,
                     sem_in0, sem_in1, sem_out0, sem_out1):
        in_bufs = (in_v0, in_v1)
        out_bufs = (out_v0, out_v1)
        in_sems = (sem_in0, sem_in1)
        out_sems = (sem_out0, sem_out1)

        wid = lax.axis_index("s") * num_cores + lax.axis_index("c")
        base = wid * rows_per_worker * N_FEAT

        def in_block(b):
            return x_hbm.at[pl.ds(base + b * block_words, block_words)]

        def out_block(b):
            return out_hbm.at[pl.ds(base + b * block_words, block_words)]

        pltpu.async_copy(in_block(0), in_v0, sem_in0)
        pltpu.async_copy(in_block(1), in_v1, sem_in1)
        pltpu.sync_copy(idx_hbm, idx_v)

        def permute_block(src, dst):
            def col_body(jb, c):
                col0 = jb * lanes
                idxv = idx_v[pl.ds(col0, lanes)]
                for r in range(BLOCK_ROWS):
                    vals = plsc.load_gather(src, [idxv + r * N_FEAT])
                    dst[pl.ds(r * N_FEAT + col0, lanes)] = vals
                return c

            lax.fori_loop(0, N_FEAT // lanes, col_body, 0, unroll=2)

        def pair_body(p, carry):
            for buf in range(2):
                b = p * 2 + buf
                pltpu.make_async_copy(in_block(b), in_bufs[buf],
                                      in_sems[buf]).wait()
                @pl.when(p > 0)
                def _():
                    pltpu.make_async_copy(out_bufs[buf], out_block(b),
                                          out_sems[buf]).wait()
                permute_block(in_bufs[buf], out_bufs[buf])
                pltpu.async_copy(out_bufs[buf], out_block(b), out_sems[buf])

                @pl.when(b + 2 < num_blocks)
                def _():
                    pltpu.async_copy(in_block(b + 2), in_bufs[buf],
                                     in_sems[buf])
            return carry

        lax.fori_loop(0, num_pairs, pair_body, 0)
        pltpu.make_async_copy(out_bufs[0], out_block(0), out_sems[0]).wait()
        pltpu.make_async_copy(out_bufs[1], out_block(1), out_sems[1]).wait()

    out_flat = permute_rows(x.reshape(-1), indices)
    return out_flat.reshape(N_ROWS, N_FEAT)


# speedup vs baseline: 1.6154x; 1.3978x over previous
import functools

import jax
import jax.numpy as jnp
from jax import lax
from jax.experimental import pallas as pl
from jax.experimental.pallas import tpu as pltpu
from jax.experimental.pallas import tpu_sc as plsc

N_ROWS = 16384
N_FEAT = 2048
BLOCK_ROWS = 8


def kernel(x, indices):
    info = plsc.get_sparse_core_info()
    num_cores, num_subcores, lanes = (
        info.num_cores, info.num_subcores, info.num_lanes)
    num_workers = num_cores * num_subcores
    rows_per_worker = N_ROWS // num_workers
    num_blocks = rows_per_worker // BLOCK_ROWS
    num_pairs = num_blocks // 2
    block_words = BLOCK_ROWS * N_FEAT
    mesh = plsc.VectorSubcoreMesh(core_axis_name="c", subcore_axis_name="s")

    @functools.partial(
        pl.kernel,
        mesh=mesh,
        compiler_params=pltpu.CompilerParams(needs_layout_passes=False),
        out_type=jax.ShapeDtypeStruct((N_ROWS * N_FEAT,), jnp.float32),
        scratch_types=[
            pltpu.VMEM((N_FEAT,), jnp.int32),
            pltpu.VMEM((block_words,), jnp.float32),
            pltpu.VMEM((block_words,), jnp.float32),
            pltpu.VMEM((block_words,), jnp.float32),
            pltpu.VMEM((block_words,), jnp.float32),
            pltpu.SemaphoreType.DMA,
            pltpu.SemaphoreType.DMA,
            pltpu.SemaphoreType.DMA,
            pltpu.SemaphoreType.DMA,
        ],
    )
    def permute_rows(x_hbm, idx_hbm, out_hbm, idx_v,
                     in_v0, in_v1, out_v0, out_v1,
                     sem_in0, sem_in1, sem_out0, sem_out1):
        in_bufs = (in_v0, in_v1)
        out_bufs = (out_v0, out_v1)
        in_sems = (sem_in0, sem_in1)
        out_sems = (sem_out0, sem_out1)

        wid = lax.axis_index("s") * num_cores + lax.axis_index("c")
        base = wid * rows_per_worker * N_FEAT

        def in_block(b):
            return x_hbm.at[pl.ds(base + b * block_words, block_words)]

        def out_block(b):
            return out_hbm.at[pl.ds(base + b * block_words, block_words)]

        pltpu.async_copy(in_block(0), in_v0, sem_in0)
        pltpu.async_copy(in_block(1), in_v1, sem_in1)
        pltpu.sync_copy(idx_hbm, idx_v)

        def permute_block(src, dst):
            def col_body(jb, c):
                col0 = jb * lanes
                idxv = idx_v[pl.ds(col0, lanes)]
                vals = [plsc.load_gather(src, [idxv + r * N_FEAT])
                        for r in range(BLOCK_ROWS)]
                for r in range(BLOCK_ROWS):
                    dst[pl.ds(r * N_FEAT + col0, lanes)] = vals[r]
                return c

            lax.fori_loop(0, N_FEAT // lanes, col_body, 0, unroll=2)

        def pair_body(p, carry):
            for buf in range(2):
                b = p * 2 + buf
                pltpu.make_async_copy(in_block(b), in_bufs[buf],
                                      in_sems[buf]).wait()
                @pl.when(p > 0)
                def _():
                    pltpu.make_async_copy(out_bufs[buf], out_block(b),
                                          out_sems[buf]).wait()
                permute_block(in_bufs[buf], out_bufs[buf])
                pltpu.async_copy(out_bufs[buf], out_block(b), out_sems[buf])

                @pl.when(b + 2 < num_blocks)
                def _():
                    pltpu.async_copy(in_block(b + 2), in_bufs[buf],
                                     in_sems[buf])
            return carry

        lax.fori_loop(0, num_pairs, pair_body, 0)
        pltpu.make_async_copy(out_bufs[0], out_block(0), out_sems[0]).wait()
        pltpu.make_async_copy(out_bufs[1], out_block(1), out_sems[1]).wait()

    out_flat = permute_rows(x.reshape(-1), indices)
    return out_flat.reshape(N_ROWS, N_FEAT)


# speedup vs baseline: 3.2998x; 2.0427x over previous
import functools

import jax
import jax.numpy as jnp
from jax import lax
from jax.experimental import pallas as pl
from jax.experimental.pallas import tpu as pltpu
from jax.experimental.pallas import tpu_sc as plsc

N_ROWS = 16384
N_FEAT = 2048
BLOCK_ROWS = 8


def kernel(x, indices):
    info = plsc.get_sparse_core_info()
    num_cores, num_subcores, lanes = (
        info.num_cores, info.num_subcores, info.num_lanes)
    num_workers = num_cores * num_subcores
    rows_per_worker = N_ROWS // num_workers
    num_blocks = rows_per_worker // BLOCK_ROWS
    num_pairs = num_blocks // 2
    mesh = plsc.VectorSubcoreMesh(core_axis_name="c", subcore_axis_name="s")

    @functools.partial(
        pl.kernel,
        mesh=mesh,
        compiler_params=pltpu.CompilerParams(needs_layout_passes=False),
        out_type=jax.ShapeDtypeStruct((N_ROWS, N_FEAT), jnp.float32),
        scratch_types=[
            pltpu.VMEM((N_FEAT,), jnp.int32),
            pltpu.VMEM((BLOCK_ROWS, N_FEAT), jnp.float32),
            pltpu.VMEM((BLOCK_ROWS, N_FEAT), jnp.float32),
            pltpu.VMEM((BLOCK_ROWS, N_FEAT), jnp.float32),
            pltpu.VMEM((BLOCK_ROWS, N_FEAT), jnp.float32),
            pltpu.SemaphoreType.DMA,
            pltpu.SemaphoreType.DMA,
            pltpu.SemaphoreType.DMA,
            pltpu.SemaphoreType.DMA,
        ],
    )
    def permute_rows(x_hbm, idx_hbm, out_hbm, idx_v,
                     in_v0, in_v1, out_v0, out_v1,
                     sem_in0, sem_in1, sem_out0, sem_out1):
        in_bufs = (in_v0, in_v1)
        out_bufs = (out_v0, out_v1)
        in_sems = (sem_in0, sem_in1)
        out_sems = (sem_out0, sem_out1)

        wid = lax.axis_index("s") * num_cores + lax.axis_index("c")
        base = wid * rows_per_worker

        def in_block(b):
            return x_hbm.at[pl.ds(base + b * BLOCK_ROWS, BLOCK_ROWS)]

        def out_block(b):
            return out_hbm.at[pl.ds(base + b * BLOCK_ROWS, BLOCK_ROWS)]

        pltpu.async_copy(in_block(0), in_v0, sem_in0)
        pltpu.async_copy(in_block(1), in_v1, sem_in1)
        pltpu.sync_copy(idx_hbm, idx_v)

        def permute_block(src, dst):
            def col_body(jb, c):
                col0 = jb * lanes
                idxv = idx_v[pl.ds(col0, lanes)]
                vals = [
                    plsc.load_gather(
                        src, [jnp.full((lanes,), r, jnp.int32), idxv])
                    for r in range(BLOCK_ROWS)
                ]
                for r in range(BLOCK_ROWS):
                    dst[r, pl.ds(col0, lanes)] = vals[r]
                return c

            lax.fori_loop(0, N_FEAT // lanes, col_body, 0, unroll=2)

        def pair_body(p, carry):
            for buf in range(2):
                b = p * 2 + buf
                pltpu.make_async_copy(in_block(b), in_bufs[buf],
                                      in_sems[buf]).wait()
                @pl.when(p > 0)
                def _():
                    pltpu.make_async_copy(out_bufs[buf], out_block(b),
                                          out_sems[buf]).wait()
                permute_block(in_bufs[buf], out_bufs[buf])
                pltpu.async_copy(out_bufs[buf], out_block(b), out_sems[buf])

                @pl.when(b + 2 < num_blocks)
                def _():
                    pltpu.async_copy(in_block(b + 2), in_bufs[buf],
                                     in_sems[buf])
            return carry

        lax.fori_loop(0, num_pairs, pair_body, 0)
        pltpu.make_async_copy(out_bufs[0], out_block(0), out_sems[0]).wait()
        pltpu.make_async_copy(out_bufs[1], out_block(1), out_sems[1]).wait()

    return permute_rows(x, indices)


# speedup vs baseline: 4.4324x; 1.3432x over previous
import functools

import jax
import jax.numpy as jnp
from jax import lax
from jax.experimental import pallas as pl
from jax.experimental.pallas import tpu as pltpu
from jax.experimental.pallas import tpu_sc as plsc

N_ROWS = 16384
N_FEAT = 2048
BLOCK_ROWS = 8


def kernel(x, indices):
    info = plsc.get_sparse_core_info()
    num_cores, num_subcores, lanes = (
        info.num_cores, info.num_subcores, info.num_lanes)
    num_workers = num_cores * num_subcores
    rows_per_worker = N_ROWS // num_workers
    num_blocks = rows_per_worker // BLOCK_ROWS
    num_pairs = num_blocks // 2
    mesh = plsc.VectorSubcoreMesh(core_axis_name="c", subcore_axis_name="s")

    @functools.partial(
        pl.kernel,
        mesh=mesh,
        compiler_params=pltpu.CompilerParams(needs_layout_passes=False),
        out_type=jax.ShapeDtypeStruct((N_ROWS, N_FEAT), jnp.float32),
        scratch_types=[
            pltpu.VMEM((N_FEAT,), jnp.int32),
            pltpu.VMEM((BLOCK_ROWS, N_FEAT), jnp.float32),
            pltpu.VMEM((BLOCK_ROWS, N_FEAT), jnp.float32),
            pltpu.VMEM((BLOCK_ROWS, N_FEAT), jnp.float32),
            pltpu.VMEM((BLOCK_ROWS, N_FEAT), jnp.float32),
            pltpu.SemaphoreType.DMA,
            pltpu.SemaphoreType.DMA,
            pltpu.SemaphoreType.DMA,
            pltpu.SemaphoreType.DMA,
        ],
    )
    def permute_rows(x_hbm, idx_hbm, out_hbm, idx_v,
                     in_v0, in_v1, out_v0, out_v1,
                     sem_in0, sem_in1, sem_out0, sem_out1):
        in_bufs = (in_v0, in_v1)
        out_bufs = (out_v0, out_v1)
        in_sems = (sem_in0, sem_in1)
        out_sems = (sem_out0, sem_out1)

        wid = lax.axis_index("s") * num_cores + lax.axis_index("c")
        base = wid * rows_per_worker

        def in_block(b):
            return x_hbm.at[pl.ds(base + b * BLOCK_ROWS, BLOCK_ROWS)]

        def out_block(b):
            return out_hbm.at[pl.ds(base + b * BLOCK_ROWS, BLOCK_ROWS)]

        pltpu.async_copy(in_block(0), in_v0, sem_in0)
        pltpu.async_copy(in_block(1), in_v1, sem_in1)
        pltpu.sync_copy(idx_hbm, idx_v)

        last_col0 = N_FEAT - lanes

        def permute_block(src, dst):
            def col_body(jb, idxv):
                next_col0 = jnp.minimum((jb + 1) * lanes, last_col0)
                idxv_next = idx_v[pl.ds(next_col0, lanes)]
                col0 = jb * lanes
                vals = [
                    plsc.load_gather(
                        src, [jnp.full((lanes,), r, jnp.int32), idxv])
                    for r in range(BLOCK_ROWS)
                ]
                for r in range(BLOCK_ROWS):
                    dst[r, pl.ds(col0, lanes)] = vals[r]
                return idxv_next

            lax.fori_loop(0, N_FEAT // lanes, col_body,
                          idx_v[pl.ds(0, lanes)], unroll=2)

        def pair_body(p, carry):
            for buf in range(2):
                b = p * 2 + buf
                pltpu.make_async_copy(in_block(b), in_bufs[buf],
                                      in_sems[buf]).wait()
                @pl.when(p > 0)
                def _():
                    pltpu.make_async_copy(out_bufs[buf], out_block(b),
                                          out_sems[buf]).wait()
                permute_block(in_bufs[buf], out_bufs[buf])
                pltpu.async_copy(out_bufs[buf], out_block(b), out_sems[buf])

                @pl.when(b + 2 < num_blocks)
                def _():
                    pltpu.async_copy(in_block(b + 2), in_bufs[buf],
                                     in_sems[buf])
            return carry

        lax.fori_loop(0, num_pairs, pair_body, 0)
        pltpu.make_async_copy(out_bufs[0], out_block(0), out_sems[0]).wait()
        pltpu.make_async_copy(out_bufs[1], out_block(1), out_sems[1]).wait()

    return permute_rows(x, indices)


# speedup vs baseline: 4.8799x; 1.1010x over previous
import functools

import jax
import jax.numpy as jnp
from jax import lax
from jax.experimental import pallas as pl
from jax.experimental.pallas import tpu as pltpu
from jax.experimental.pallas import tpu_sc as plsc

N_ROWS = 16384
N_FEAT = 2048
BLOCK_ROWS = 8


def kernel(x, indices):
    info = plsc.get_sparse_core_info()
    num_cores, num_subcores, lanes = (
        info.num_cores, info.num_subcores, info.num_lanes)
    num_workers = num_cores * num_subcores
    rows_per_worker = N_ROWS // num_workers
    num_blocks = rows_per_worker // BLOCK_ROWS
    num_pairs = num_blocks // 2
    mesh = plsc.VectorSubcoreMesh(core_axis_name="c", subcore_axis_name="s")

    @functools.partial(
        pl.kernel,
        mesh=mesh,
        compiler_params=pltpu.CompilerParams(needs_layout_passes=False),
        out_type=jax.ShapeDtypeStruct((N_ROWS, N_FEAT), jnp.float32),
        scratch_types=[
            pltpu.VMEM((N_FEAT,), jnp.int32),
            pltpu.VMEM((BLOCK_ROWS, N_FEAT), jnp.float32),
            pltpu.VMEM((BLOCK_ROWS, N_FEAT), jnp.float32),
            pltpu.VMEM((BLOCK_ROWS, N_FEAT), jnp.float32),
            pltpu.VMEM((BLOCK_ROWS, N_FEAT), jnp.float32),
            pltpu.SemaphoreType.DMA,
            pltpu.SemaphoreType.DMA,
            pltpu.SemaphoreType.DMA,
            pltpu.SemaphoreType.DMA,
        ],
    )
    def permute_rows(x_hbm, idx_hbm, out_hbm, idx_v,
                     in_v0, in_v1, out_v0, out_v1,
                     sem_in0, sem_in1, sem_out0, sem_out1):
        in_bufs = (in_v0, in_v1)
        out_bufs = (out_v0, out_v1)
        in_sems = (sem_in0, sem_in1)
        out_sems = (sem_out0, sem_out1)

        wid = lax.axis_index("s") * num_cores + lax.axis_index("c")
        base = wid * rows_per_worker

        def in_block(b):
            return x_hbm.at[pl.ds(base + b * BLOCK_ROWS, BLOCK_ROWS)]

        def out_block(b):
            return out_hbm.at[pl.ds(base + b * BLOCK_ROWS, BLOCK_ROWS)]

        pltpu.async_copy(in_block(0), in_v0, sem_in0)
        pltpu.async_copy(in_block(1), in_v1, sem_in1)
        pltpu.sync_copy(idx_hbm, idx_v)

        last_col0 = N_FEAT - lanes
        num_cols = N_FEAT // lanes

        def permute_block(src, dst):
            def gather8(idxv):
                return [
                    plsc.load_gather(
                        src, [jnp.full((lanes,), r, jnp.int32), idxv])
                    for r in range(BLOCK_ROWS)
                ]

            def store8(col0, vals):
                for r in range(BLOCK_ROWS):
                    dst[r, pl.ds(col0, lanes)] = vals[r]

            idxv0 = idx_v[pl.ds(0, lanes)]
            vals0 = gather8(idxv0)

            def col_body(jb, carry):
                idxv, prev_vals = carry
                next_col0 = jnp.minimum((jb + 1) * lanes, last_col0)
                idxv_next = idx_v[pl.ds(next_col0, lanes)]
                vals = gather8(idxv)
                store8((jb - 1) * lanes, prev_vals)
                return idxv_next, vals

            _, last_vals = lax.fori_loop(
                1, num_cols, col_body,
                (idx_v[pl.ds(lanes, lanes)], vals0), unroll=2)
            store8(last_col0, last_vals)

        def pair_body(p, carry):
            for buf in range(2):
                b = p * 2 + buf
                pltpu.make_async_copy(in_block(b), in_bufs[buf],
                                      in_sems[buf]).wait()
                @pl.when(p > 0)
                def _():
                    pltpu.make_async_copy(out_bufs[buf], out_block(b),
                                          out_sems[buf]).wait()
                permute_block(in_bufs[buf], out_bufs[buf])
                pltpu.async_copy(out_bufs[buf], out_block(b), out_sems[buf])

                @pl.when(b + 2 < num_blocks)
                def _():
                    pltpu.async_copy(in_block(b + 2), in_bufs[buf],
                                     in_sems[buf])
            return carry

        lax.fori_loop(0, num_pairs, pair_body, 0)
        pltpu.make_async_copy(out_bufs[0], out_block(0), out_sems[0]).wait()
        pltpu.make_async_copy(out_bufs[1], out_block(1), out_sems[1]).wait()

    return permute_rows(x, indices)


# speedup vs baseline: 5.6973x; 1.1675x over previous
import functools

import jax
import jax.numpy as jnp
from jax import lax
from jax.experimental import pallas as pl
from jax.experimental.pallas import tpu as pltpu
from jax.experimental.pallas import tpu_sc as plsc

N_ROWS = 16384
N_FEAT = 2048
IN_ROWS = 16
OUT_ROWS = 8


def kernel(x, indices):
    info = plsc.get_sparse_core_info()
    num_cores, num_subcores, lanes = (
        info.num_cores, info.num_subcores, info.num_lanes)
    num_workers = num_cores * num_subcores
    rows_per_worker = N_ROWS // num_workers
    num_blocks = rows_per_worker // IN_ROWS
    mesh = plsc.VectorSubcoreMesh(core_axis_name="c", subcore_axis_name="s")

    @functools.partial(
        pl.kernel,
        mesh=mesh,
        compiler_params=pltpu.CompilerParams(needs_layout_passes=False),
        out_type=jax.ShapeDtypeStruct((N_ROWS, N_FEAT), jnp.float32),
        scratch_types=[
            pltpu.VMEM((N_FEAT,), jnp.int32),
            pltpu.VMEM((IN_ROWS, N_FEAT), jnp.float32),
            pltpu.VMEM((IN_ROWS, N_FEAT), jnp.float32),
            pltpu.VMEM((OUT_ROWS, N_FEAT), jnp.float32),
            pltpu.VMEM((OUT_ROWS, N_FEAT), jnp.float32),
            pltpu.SemaphoreType.DMA,
            pltpu.SemaphoreType.DMA,
            pltpu.SemaphoreType.DMA,
            pltpu.SemaphoreType.DMA,
        ],
    )
    def permute_rows(x_hbm, idx_hbm, out_hbm, idx_v,
                     in_v0, in_v1, out_v0, out_v1,
                     sem_in0, sem_in1, sem_out0, sem_out1):
        in_bufs = (in_v0, in_v1)
        out_bufs = (out_v0, out_v1)
        in_sems = (sem_in0, sem_in1)
        out_sems = (sem_out0, sem_out1)

        wid = lax.axis_index("s") * num_cores + lax.axis_index("c")
        base = wid * rows_per_worker

        def in_block(b):
            return x_hbm.at[pl.ds(base + b * IN_ROWS, IN_ROWS)]

        def out_block(o):
            return out_hbm.at[pl.ds(base + o * OUT_ROWS, OUT_ROWS)]

        for j in range(2):
            pltpu.async_copy(in_block(j), in_bufs[j], in_sems[j])
        pltpu.sync_copy(idx_hbm, idx_v)

        num_cols = N_FEAT // lanes

        def permute_half(src, row0, dst):
            def gather8(idxv):
                return [
                    plsc.load_gather(
                        src, [jnp.full((lanes,), row0 + r, jnp.int32), idxv])
                    for r in range(OUT_ROWS)
                ]

            def store8(col0, vals):
                for r in range(OUT_ROWS):
                    dst[r, pl.ds(col0, lanes)] = vals[r]

            @plsc.parallel_loop(0, num_cols, unroll=4)
            def _(jb):
                col0 = jb * lanes
                store8(col0, gather8(idx_v[pl.ds(col0, lanes)]))

        def block_body(g, carry):
            for j in range(2):
                b = g * 2 + j
                pltpu.make_async_copy(in_block(b), in_bufs[j],
                                      in_sems[j]).wait()
                for half in range(2):
                    o = 2 * b + half
                    ob = half
                    @pl.when(o >= 2)
                    def _():
                        pltpu.make_async_copy(out_bufs[ob], out_block(o),
                                              out_sems[ob]).wait()
                    permute_half(in_bufs[j], half * OUT_ROWS, out_bufs[ob])
                    pltpu.async_copy(out_bufs[ob], out_block(o), out_sems[ob])

                @pl.when(b + 2 < num_blocks)
                def _():
                    pltpu.async_copy(in_block(b + 2), in_bufs[j],
                                     in_sems[j])
            return carry

        lax.fori_loop(0, num_blocks // 2, block_body, 0)
        for j in range(2):
            pltpu.make_async_copy(out_bufs[j], out_block(0), out_sems[j]).wait()

    return permute_rows(x, indices)


# speedup vs baseline: 5.7425x; 1.0079x over previous
import functools

import jax
import jax.numpy as jnp
from jax import lax
from jax.experimental import pallas as pl
from jax.experimental.pallas import tpu as pltpu
from jax.experimental.pallas import tpu_sc as plsc

N_ROWS = 16384
N_FEAT = 2048
BLOCK_ROWS = 8


def kernel(x, indices):
    info = plsc.get_sparse_core_info()
    num_cores, num_subcores, lanes = (
        info.num_cores, info.num_subcores, info.num_lanes)
    num_workers = num_cores * num_subcores
    rows_per_worker = N_ROWS // num_workers
    num_blocks = rows_per_worker // BLOCK_ROWS
    num_pairs = num_blocks // 2
    mesh = plsc.VectorSubcoreMesh(core_axis_name="c", subcore_axis_name="s")

    @functools.partial(
        pl.kernel,
        mesh=mesh,
        compiler_params=pltpu.CompilerParams(needs_layout_passes=False,
                                             skip_device_barrier=True),
        out_type=jax.ShapeDtypeStruct((N_ROWS, N_FEAT), jnp.float32),
        scratch_types=[
            pltpu.VMEM((N_FEAT,), jnp.int32),
            pltpu.VMEM((BLOCK_ROWS, N_FEAT), jnp.float32),
            pltpu.VMEM((BLOCK_ROWS, N_FEAT), jnp.float32),
            pltpu.VMEM((BLOCK_ROWS, N_FEAT), jnp.float32),
            pltpu.VMEM((BLOCK_ROWS, N_FEAT), jnp.float32),
            pltpu.VMEM((BLOCK_ROWS, N_FEAT), jnp.float32),
            pltpu.VMEM((BLOCK_ROWS, N_FEAT), jnp.float32),
            pltpu.SemaphoreType.DMA,
            pltpu.SemaphoreType.DMA,
            pltpu.SemaphoreType.DMA,
            pltpu.SemaphoreType.DMA,
            pltpu.SemaphoreType.DMA,
            pltpu.SemaphoreType.DMA,
        ],
    )
    def permute_rows(x_hbm, idx_hbm, out_hbm, idx_v,
                     in_v0, in_v1, in_v2, out_v0, out_v1, out_v2,
                     sem_in0, sem_in1, sem_in2, sem_out0, sem_out1,
                     sem_out2):
        in_bufs = (in_v0, in_v1, in_v2)
        out_bufs = (out_v0, out_v1, out_v2)
        in_sems = (sem_in0, sem_in1, sem_in2)
        out_sems = (sem_out0, sem_out1, sem_out2)

        wid = lax.axis_index("s") * num_cores + lax.axis_index("c")
        base = wid * rows_per_worker

        def in_block(b):
            return x_hbm.at[pl.ds(base + b * BLOCK_ROWS, BLOCK_ROWS)]

        def out_block(b):
            return out_hbm.at[pl.ds(base + b * BLOCK_ROWS, BLOCK_ROWS)]

        for j in range(3):
            pltpu.async_copy(in_block(j), in_bufs[j], in_sems[j])
        pltpu.sync_copy(idx_hbm, idx_v)

        last_col0 = N_FEAT - lanes
        num_cols = N_FEAT // lanes

        def permute_block(src, dst):
            def gather8(idxv):
                return [
                    plsc.load_gather(
                        src, [jnp.full((lanes,), r, jnp.int32), idxv])
                    for r in range(BLOCK_ROWS)
                ]

            def store8(col0, vals):
                for r in range(BLOCK_ROWS):
                    dst[r, pl.ds(col0, lanes)] = vals[r]

            @plsc.parallel_loop(0, num_cols, unroll=4)
            def _(jb):
                col0 = jb * lanes
                store8(col0, gather8(idx_v[pl.ds(col0, lanes)]))

        def do_block(b, ring):
            pltpu.make_async_copy(in_block(b), in_bufs[ring],
                                  in_sems[ring]).wait()
            @pl.when(b >= 3)
            def _():
                pltpu.make_async_copy(out_bufs[ring], out_block(b),
                                      out_sems[ring]).wait()
            permute_block(in_bufs[ring], out_bufs[ring])
            pltpu.async_copy(out_bufs[ring], out_block(b), out_sems[ring])

            @pl.when(b + 3 < num_blocks)
            def _():
                pltpu.async_copy(in_block(b + 3), in_bufs[ring],
                                 in_sems[ring])

        def group_body(g, carry):
            for j in range(3):
                do_block(g * 3 + j, j)
            return carry

        lax.fori_loop(0, (num_blocks - 1) // 3, group_body, 0)
        do_block(num_blocks - 1, 0)
        for j in range(3):
            pltpu.make_async_copy(out_bufs[j], out_block(0), out_sems[j]).wait()

    return permute_rows(x, indices)
